# Initial kernel scaffold; baseline (speedup 1.0000x reference)
import jax
import jax.numpy as jnp
from jax.experimental import pallas as pl


def kernel(coord, params):
    raise NotImplementedError("write your pallas kernel here")


# speedup vs baseline: 11.3900x; 11.3900x over previous
import functools

import jax
import jax.numpy as jnp
import numpy as np
from jax.experimental import pallas as pl

_EPS = 1e-5
_F32 = jnp.float32
_INTERPRET = False


def _fps_body(x_ref, out_ref, *, n_sample):
    b, _, n = x_ref.shape
    xx = x_ref[:, 0, :]
    yy = x_ref[:, 1, :]
    zz = x_ref[:, 2, :]
    col = jax.lax.broadcasted_iota(jnp.int32, (b, n), 1)

    def step(i, state):
        dist, far = state
        sel = col == far
        cx = jnp.sum(jnp.where(sel, xx, 0.0), axis=1, keepdims=True)
        cy = jnp.sum(jnp.where(sel, yy, 0.0), axis=1, keepdims=True)
        cz = jnp.sum(jnp.where(sel, zz, 0.0), axis=1, keepdims=True)
        c3 = jnp.concatenate([cx, cy, cz], axis=1)
        out_ref[:, pl.ds(i, 1), :] = c3.reshape(b, 1, 3)
        dx = xx - cx
        dy = yy - cy
        dz = zz - cz
        d = dx * dx + dy * dy + dz * dz
        dist = jnp.minimum(dist, d)
        m = jnp.max(dist, axis=1, keepdims=True)
        far = jnp.min(jnp.where(dist == m, col, n), axis=1, keepdims=True)
        return dist, far

    dist0 = jnp.full((b, n), 1e10, dtype=_F32)
    far0 = jnp.zeros((b, 1), dtype=jnp.int32)
    jax.lax.fori_loop(0, n_sample, step, (dist0, far0))


def _cumsum_lanes(mf, st, n, ch=128):
    c = n // ch
    x = mf.reshape(st * c, ch)
    r_i = jax.lax.broadcasted_iota(jnp.int32, (ch, ch), 0)
    c_i = jax.lax.broadcasted_iota(jnp.int32, (ch, ch), 1)
    tri = (r_i <= c_i).astype(_F32)
    loc = jax.lax.dot_general(x, tri, (((1,), (0,)), ((), ())),
                              preferred_element_type=_F32)
    loc = loc.reshape(st, c, ch)
    tot = loc[:, :, ch - 1]
    r2_i = jax.lax.broadcasted_iota(jnp.int32, (c, c), 0)
    c2_i = jax.lax.broadcasted_iota(jnp.int32, (c, c), 1)
    tri_s = (r2_i < c2_i).astype(_F32)
    exc = jax.lax.dot_general(tot, tri_s, (((1,), (0,)), ((), ())),
                              preferred_element_type=_F32)
    cum = loc + exc.reshape(st, c, 1)
    return cum.reshape(st, n)


def _select_matrix(cum, mask, st, k, n):
    cnt = cum[:, n - 1:n].reshape(st, 1, 1)
    ik = jax.lax.broadcasted_iota(jnp.int32, (st, k, 1), 1).astype(_F32)
    tgt = jnp.where(ik < cnt, ik + 1.0, 1.0)
    sel = (cum.reshape(st, 1, n) == tgt) & mask.reshape(st, 1, n)
    return jnp.where(sel, 1.0, 0.0).reshape(st * k, n)


def _mlp(h, layers):
    for (w, bb, gs, be) in layers:
        z = jax.lax.dot_general(h, w[...], (((1,), (1,)), ((), ())),
                                preferred_element_type=_F32)
        z = (z + bb[...]) * gs[...] + be[...]
        h = jnp.maximum(z, 0.0)
    return h


def _unpack_layers(refs):
    return [tuple(refs[i:i + 4]) for i in range(0, len(refs), 4)]


def _sa1_body(pts_ref, cen_ref, *rest, st, k, r2, dout):
    out_ref = rest[-1]
    layers = _unpack_layers(rest[:-1])
    x = pts_ref[0]
    c = cen_ref[0]
    n = x.shape[1]
    dx = c[:, 0:1] - x[0:1, :]
    dy = c[:, 1:2] - x[1:2, :]
    dz = c[:, 2:3] - x[2:3, :]
    dsq = dx * dx + dy * dy + dz * dz
    mask = dsq <= r2
    cum = _cumsum_lanes(mask.astype(_F32), st, n)
    selm = _select_matrix(cum, mask, st, k, n)
    gx = jax.lax.dot_general(selm, x, (((1,), (1,)), ((), ())),
                             preferred_element_type=_F32)
    crep = jnp.broadcast_to(c.reshape(st, 1, 3), (st, k, 3)).reshape(st * k, 3)
    h = _mlp(gx - crep, layers)
    out_ref[0] = jnp.max(h.reshape(st, k, dout), axis=1)


def _sa2_body(pts_ref, cen_ref, feat_ref, *rest, st, k, r2, dout):
    out_ref = rest[-1]
    layers = _unpack_layers(rest[:-1])
    x = pts_ref[0]
    c = cen_ref[0]
    fb = feat_ref[0]
    n = x.shape[1]
    dx = c[:, 0:1] - x[0:1, :]
    dy = c[:, 1:2] - x[1:2, :]
    dz = c[:, 2:3] - x[2:3, :]
    dsq = dx * dx + dy * dy + dz * dz
    mask = dsq <= r2
    cum = _cumsum_lanes(mask.astype(_F32), st, n)
    selm = _select_matrix(cum, mask, st, k, n)
    gx = jax.lax.dot_general(selm, x, (((1,), (1,)), ((), ())),
                             preferred_element_type=_F32)
    gf = jax.lax.dot_general(selm, fb, (((1,), (0,)), ((), ())),
                             preferred_element_type=_F32)
    crep = jnp.broadcast_to(c.reshape(st, 1, 3), (st, k, 3)).reshape(st * k, 3)
    h = jnp.concatenate([gx - crep, gf], axis=1)
    h = _mlp(h, layers)
    out_ref[0] = jnp.max(h.reshape(st, k, dout), axis=1)


def _sa3_body(cen_ref, feat_ref, *rest):
    out_ref = rest[-1]
    layers = _unpack_layers(rest[:-1])
    h = jnp.concatenate([cen_ref[0], feat_ref[0]], axis=1)
    h = _mlp(h, layers)
    out_ref[...] = jnp.max(h, axis=0, keepdims=True).reshape(1, 1, -1)


def _head_body(p_ref, *rest):
    out_ref = rest[-1]
    wo = rest[-3]
    bo = rest[-2]
    layers = _unpack_layers(rest[:-3])
    h = _mlp(p_ref[...], layers)
    logits = jax.lax.dot_general(h, wo[...], (((1,), (1,)), ((), ())),
                                 preferred_element_type=_F32) + bo[...]
    m = jnp.max(logits, axis=1, keepdims=True)
    shifted = logits - m
    lse = jnp.log(jnp.sum(jnp.exp(shifted), axis=1, keepdims=True))
    out_ref[...] = shifted - lse


def _prep_layers(layers):
    s = jnp.sqrt(jnp.asarray(1.0 + _EPS, dtype=_F32))
    out = []
    for (w, b, g, be) in layers:
        out.extend([w, b.reshape(1, -1), (g / s).reshape(1, -1),
                    be.reshape(1, -1)])
    return out


def _const_specs(arrs):
    return [pl.BlockSpec(a.shape, lambda *idx, nd=a.ndim: (0,) * nd)
            for a in arrs]


def _run_fps(x, n_sample):
    b = x.shape[0]
    return pl.pallas_call(
        functools.partial(_fps_body, n_sample=n_sample),
        out_shape=jax.ShapeDtypeStruct((b, n_sample, 3), _F32),
        interpret=_INTERPRET,
    )(x)


def _run_sa(body, pts, cen, feats, wflat, *, st, k, r2, dout):
    b, _, n = pts.shape
    s = cen.shape[1]
    args = [pts, cen] + ([feats] if feats is not None else []) + wflat
    in_specs = [
        pl.BlockSpec((1, 3, n), lambda bi, si: (bi, 0, 0)),
        pl.BlockSpec((1, st, 3), lambda bi, si: (bi, si, 0)),
    ]
    if feats is not None:
        df = feats.shape[2]
        in_specs.append(pl.BlockSpec((1, n, df), lambda bi, si: (bi, 0, 0)))
    in_specs += _const_specs(wflat)
    return pl.pallas_call(
        functools.partial(body, st=st, k=k, r2=r2, dout=dout),
        grid=(b, s // st),
        in_specs=in_specs,
        out_specs=pl.BlockSpec((1, st, dout), lambda bi, si: (bi, si, 0)),
        out_shape=jax.ShapeDtypeStruct((b, s, dout), _F32),
        interpret=_INTERPRET,
    )(*args)


def kernel(coord, params):
    coord = coord.astype(_F32)
    b, _, n = coord.shape

    sa1 = _prep_layers(params["sa1"])
    sa2 = _prep_layers(params["sa2"])
    sa3 = _prep_layers(params["sa3"])
    head = _prep_layers(params["head"])
    wo, bo = params["head_out"]
    bo2 = bo.reshape(1, -1)

    nx1 = _run_fps(coord, 512)
    f1 = _run_sa(_sa1_body, coord, nx1, None, sa1,
                 st=16, k=32, r2=np.float32(0.2 * 0.2), dout=128)

    nx1_t = jnp.transpose(nx1, (0, 2, 1))
    nx2 = _run_fps(nx1_t, 128)
    f2 = _run_sa(_sa2_body, nx1_t, nx2, f1, sa2,
                 st=32, k=64, r2=np.float32(0.4 * 0.4), dout=256)

    sa3_specs = ([pl.BlockSpec((1, 128, 3), lambda bi: (bi, 0, 0)),
                  pl.BlockSpec((1, 128, 256), lambda bi: (bi, 0, 0))]
                 + _const_specs(sa3))
    pooled = pl.pallas_call(
        _sa3_body,
        grid=(b,),
        in_specs=sa3_specs,
        out_specs=pl.BlockSpec((1, 1, 1024), lambda bi: (bi, 0, 0)),
        out_shape=jax.ShapeDtypeStruct((b, 1, 1024), _F32),
        interpret=_INTERPRET,
    )(nx2, f2, *sa3)
    pooled = pooled.reshape(b, 1024)

    logp = pl.pallas_call(
        _head_body,
        out_shape=jax.ShapeDtypeStruct((b, 40), _F32),
        interpret=_INTERPRET,
    )(pooled, *head, wo, bo2)

    new_xyz3 = jnp.zeros((b, 3, 1), dtype=coord.dtype)
    return logp, new_xyz3


# speedup vs baseline: 14.5614x; 1.2784x over previous
import functools

import jax
import jax.numpy as jnp
import numpy as np
from jax.experimental import pallas as pl

_EPS = 1e-5
_F32 = jnp.float32
_INTERPRET = False


def _fps_body(x_ref, out_ref, *, n_sample):
    b, _, n = x_ref.shape
    xx = x_ref[:, 0, :]
    yy = x_ref[:, 1, :]
    zz = x_ref[:, 2, :]
    col = jax.lax.broadcasted_iota(jnp.int32, (b, n), 1)

    def step(i, state):
        dist, far = state
        sel = col == far
        cx = jnp.sum(jnp.where(sel, xx, 0.0), axis=1, keepdims=True)
        cy = jnp.sum(jnp.where(sel, yy, 0.0), axis=1, keepdims=True)
        cz = jnp.sum(jnp.where(sel, zz, 0.0), axis=1, keepdims=True)
        c3 = jnp.concatenate([cx, cy, cz], axis=1)
        out_ref[:, pl.ds(i, 1), :] = c3.reshape(b, 1, 3)
        dx = xx - cx
        dy = yy - cy
        dz = zz - cz
        d = dx * dx + dy * dy + dz * dz
        dist = jnp.minimum(dist, d)
        m = jnp.max(dist, axis=1, keepdims=True)
        far = jnp.min(jnp.where(dist == m, col, n), axis=1, keepdims=True)
        return dist, far

    dist0 = jnp.full((b, n), 1e10, dtype=_F32)
    far0 = jnp.zeros((b, 1), dtype=jnp.int32)
    jax.lax.fori_loop(0, n_sample, step, (dist0, far0))


def _cumsum_lanes(mf, st, n, ch=128):
    c = n // ch
    x = mf.reshape(st * c, ch)
    r_i = jax.lax.broadcasted_iota(jnp.int32, (ch, ch), 0)
    c_i = jax.lax.broadcasted_iota(jnp.int32, (ch, ch), 1)
    tri = (r_i <= c_i).astype(_F32)
    loc = jax.lax.dot_general(x, tri, (((1,), (0,)), ((), ())),
                              preferred_element_type=_F32)
    loc = loc.reshape(st, c, ch)
    tot = loc[:, :, ch - 1]
    r2_i = jax.lax.broadcasted_iota(jnp.int32, (c, c), 0)
    c2_i = jax.lax.broadcasted_iota(jnp.int32, (c, c), 1)
    tri_s = (r2_i < c2_i).astype(_F32)
    exc = jax.lax.dot_general(tot, tri_s, (((1,), (0,)), ((), ())),
                              preferred_element_type=_F32)
    cum = loc + exc.reshape(st, c, 1)
    return cum.reshape(st, n)


def _select_matrix(cum, mask, st, k, n):
    cumm = jnp.where(mask, cum, -1.0)
    cnt = jnp.max(cumm, axis=1, keepdims=True).reshape(st, 1, 1)
    ik = jax.lax.broadcasted_iota(jnp.int32, (st, k, 1), 1).astype(_F32)
    tgt = jnp.where(ik < cnt, ik + 1.0, 1.0)
    sel = cumm.reshape(st, 1, n) == tgt
    return jnp.where(sel, 1.0, 0.0).reshape(st * k, n)


def _mlp(h, layers):
    for (w, bb, gs, be) in layers:
        z = jax.lax.dot_general(h, w[...], (((1,), (1,)), ((), ())),
                                preferred_element_type=_F32)
        z = (z + bb[...]) * gs[...] + be[...]
        h = jnp.maximum(z, 0.0)
    return h


def _unpack_layers(refs):
    return [tuple(refs[i:i + 4]) for i in range(0, len(refs), 4)]


def _sa1_body(pts_ref, cen_ref, *rest, st, k, r2, dout):
    out_ref = rest[-1]
    layers = _unpack_layers(rest[:-1])
    x = pts_ref[0]
    c = cen_ref[0]
    n = x.shape[1]
    dx = c[:, 0:1] - x[0:1, :]
    dy = c[:, 1:2] - x[1:2, :]
    dz = c[:, 2:3] - x[2:3, :]
    dsq = dx * dx + dy * dy + dz * dz
    mask = dsq <= r2
    cum = _cumsum_lanes(mask.astype(_F32), st, n)
    selm = _select_matrix(cum, mask, st, k, n)
    (w1, b1, g1, e1) = layers[0]
    p1 = jax.lax.dot_general(x, w1[...], (((0,), (1,)), ((), ())),
                             preferred_element_type=_F32)
    g1x = jax.lax.dot_general(selm, p1, (((1,), (0,)), ((), ())),
                              preferred_element_type=_F32)
    c1 = jax.lax.dot_general(c, w1[...], (((1,), (1,)), ((), ())),
                             preferred_element_type=_F32)
    d1 = c1.shape[1]
    c1r = jnp.broadcast_to(c1.reshape(st, 1, d1),
                           (st, k, d1)).reshape(st * k, d1)
    z1 = (g1x - c1r + b1[...]) * g1[...] + e1[...]
    h = _mlp(jnp.maximum(z1, 0.0), layers[1:])
    out_ref[0] = jnp.max(h.reshape(st, k, dout), axis=1)


def _sa2_body(pts_ref, cen_ref, feat_ref, *rest, st, k, r2, dout):
    out_ref = rest[-1]
    layers = _unpack_layers(rest[:-1])
    x = pts_ref[0]
    c = cen_ref[0]
    fb = feat_ref[0]
    n = x.shape[1]
    dx = c[:, 0:1] - x[0:1, :]
    dy = c[:, 1:2] - x[1:2, :]
    dz = c[:, 2:3] - x[2:3, :]
    dsq = dx * dx + dy * dy + dz * dz
    mask = dsq <= r2
    cum = _cumsum_lanes(mask.astype(_F32), st, n)
    selm = _select_matrix(cum, mask, st, k, n)
    gx = jax.lax.dot_general(selm, x, (((1,), (1,)), ((), ())),
                             preferred_element_type=_F32)
    gf = jax.lax.dot_general(selm, fb, (((1,), (0,)), ((), ())),
                             preferred_element_type=_F32)
    crep = jnp.broadcast_to(c.reshape(st, 1, 3), (st, k, 3)).reshape(st * k, 3)
    h = jnp.concatenate([gx - crep, gf], axis=1)
    h = _mlp(h, layers)
    out_ref[0] = jnp.max(h.reshape(st, k, dout), axis=1)


def _sa3_body(cen_ref, feat_ref, *rest):
    out_ref = rest[-1]
    layers = _unpack_layers(rest[:-1])
    h = jnp.concatenate([cen_ref[0], feat_ref[0]], axis=1)
    h = _mlp(h, layers)
    out_ref[...] = jnp.max(h, axis=0, keepdims=True).reshape(1, 1, -1)


def _head_body(p_ref, *rest):
    out_ref = rest[-1]
    wo = rest[-3]
    bo = rest[-2]
    layers = _unpack_layers(rest[:-3])
    h = _mlp(p_ref[...], layers)
    logits = jax.lax.dot_general(h, wo[...], (((1,), (1,)), ((), ())),
                                 preferred_element_type=_F32) + bo[...]
    m = jnp.max(logits, axis=1, keepdims=True)
    shifted = logits - m
    lse = jnp.log(jnp.sum(jnp.exp(shifted), axis=1, keepdims=True))
    out_ref[...] = shifted - lse


def _prep_layers(layers):
    s = jnp.sqrt(jnp.asarray(1.0 + _EPS, dtype=_F32))
    out = []
    for (w, b, g, be) in layers:
        out.extend([w, b.reshape(1, -1), (g / s).reshape(1, -1),
                    be.reshape(1, -1)])
    return out


def _const_specs(arrs):
    return [pl.BlockSpec(a.shape, lambda *idx, nd=a.ndim: (0,) * nd)
            for a in arrs]


def _run_fps(x, n_sample):
    b = x.shape[0]
    return pl.pallas_call(
        functools.partial(_fps_body, n_sample=n_sample),
        out_shape=jax.ShapeDtypeStruct((b, n_sample, 3), _F32),
        interpret=_INTERPRET,
    )(x)


def _run_sa(body, pts, cen, feats, wflat, *, st, k, r2, dout):
    b, _, n = pts.shape
    s = cen.shape[1]
    args = [pts, cen] + ([feats] if feats is not None else []) + wflat
    in_specs = [
        pl.BlockSpec((1, 3, n), lambda bi, si: (bi, 0, 0)),
        pl.BlockSpec((1, st, 3), lambda bi, si: (bi, si, 0)),
    ]
    if feats is not None:
        df = feats.shape[2]
        in_specs.append(pl.BlockSpec((1, n, df), lambda bi, si: (bi, 0, 0)))
    in_specs += _const_specs(wflat)
    return pl.pallas_call(
        functools.partial(body, st=st, k=k, r2=r2, dout=dout),
        grid=(b, s // st),
        in_specs=in_specs,
        out_specs=pl.BlockSpec((1, st, dout), lambda bi, si: (bi, si, 0)),
        out_shape=jax.ShapeDtypeStruct((b, s, dout), _F32),
        interpret=_INTERPRET,
    )(*args)


def kernel(coord, params):
    coord = coord.astype(_F32)
    b, _, n = coord.shape

    sa1 = _prep_layers(params["sa1"])
    sa2 = _prep_layers(params["sa2"])
    sa3 = _prep_layers(params["sa3"])
    head = _prep_layers(params["head"])
    wo, bo = params["head_out"]
    bo2 = bo.reshape(1, -1)

    nx1 = _run_fps(coord, 512)
    f1 = _run_sa(_sa1_body, coord, nx1, None, sa1,
                 st=32, k=32, r2=np.float32(0.2 * 0.2), dout=128)

    nx1_t = jnp.transpose(nx1, (0, 2, 1))
    nx2 = _run_fps(nx1_t, 128)
    f2 = _run_sa(_sa2_body, nx1_t, nx2, f1, sa2,
                 st=64, k=64, r2=np.float32(0.4 * 0.4), dout=256)

    sa3_specs = ([pl.BlockSpec((1, 128, 3), lambda bi: (bi, 0, 0)),
                  pl.BlockSpec((1, 128, 256), lambda bi: (bi, 0, 0))]
                 + _const_specs(sa3))
    pooled = pl.pallas_call(
        _sa3_body,
        grid=(b,),
        in_specs=sa3_specs,
        out_specs=pl.BlockSpec((1, 1, 1024), lambda bi: (bi, 0, 0)),
        out_shape=jax.ShapeDtypeStruct((b, 1, 1024), _F32),
        interpret=_INTERPRET,
    )(nx2, f2, *sa3)
    pooled = pooled.reshape(b, 1024)

    logp = pl.pallas_call(
        _head_body,
        out_shape=jax.ShapeDtypeStruct((b, 40), _F32),
        interpret=_INTERPRET,
    )(pooled, *head, wo, bo2)

    new_xyz3 = jnp.zeros((b, 3, 1), dtype=coord.dtype)
    return logp, new_xyz3


# speedup vs baseline: 16.7013x; 1.1470x over previous
import functools

import jax
import jax.numpy as jnp
import numpy as np
from jax.experimental import pallas as pl

_EPS = 1e-5
_F32 = jnp.float32
_INTERPRET = False


def _fps_body(x_ref, out_ref, *, n_sample):
    b, _, n = x_ref.shape
    xx = x_ref[:, 0, :]
    yy = x_ref[:, 1, :]
    zz = x_ref[:, 2, :]
    col = jax.lax.broadcasted_iota(jnp.int32, (b, n), 1)

    def step(i, state):
        dist, far = state
        sel = col == far
        cx = jnp.sum(jnp.where(sel, xx, 0.0), axis=1, keepdims=True)
        cy = jnp.sum(jnp.where(sel, yy, 0.0), axis=1, keepdims=True)
        cz = jnp.sum(jnp.where(sel, zz, 0.0), axis=1, keepdims=True)
        c3 = jnp.concatenate([cx, cy, cz], axis=1)
        out_ref[:, pl.ds(i, 1), :] = c3.reshape(b, 1, 3)
        dx = xx - cx
        dy = yy - cy
        dz = zz - cz
        d = dx * dx + dy * dy + dz * dz
        dist = jnp.minimum(dist, d)
        m = jnp.max(dist, axis=1, keepdims=True)
        far = jnp.min(jnp.where(dist == m, col, n), axis=1, keepdims=True)
        return dist, far

    dist0 = jnp.full((b, n), 1e10, dtype=_F32)
    far0 = jnp.zeros((b, 1), dtype=jnp.int32)
    jax.lax.fori_loop(0, n_sample, step, (dist0, far0))


def _cumsum_lanes(mf, st, n, ch=128):
    c = n // ch
    x = mf.reshape(st * c, ch)
    r_i = jax.lax.broadcasted_iota(jnp.int32, (ch, ch), 0)
    c_i = jax.lax.broadcasted_iota(jnp.int32, (ch, ch), 1)
    tri = (r_i <= c_i).astype(_F32)
    loc = jax.lax.dot_general(x, tri, (((1,), (0,)), ((), ())),
                              preferred_element_type=_F32)
    loc = loc.reshape(st, c, ch)
    tot = loc[:, :, ch - 1]
    r2_i = jax.lax.broadcasted_iota(jnp.int32, (c, c), 0)
    c2_i = jax.lax.broadcasted_iota(jnp.int32, (c, c), 1)
    tri_s = (r2_i < c2_i).astype(_F32)
    exc = jax.lax.dot_general(tot, tri_s, (((1,), (0,)), ((), ())),
                              preferred_element_type=_F32)
    cum = loc + exc.reshape(st, c, 1)
    return cum.reshape(st, n)


def _select_matrix(cum, mask, st, k, n):
    cumm = jnp.where(mask, cum, -1.0)
    cnt = jnp.max(cumm, axis=1, keepdims=True).reshape(st, 1, 1)
    ik = jax.lax.broadcasted_iota(jnp.int32, (st, k, 1), 1).astype(_F32)
    tgt = jnp.where(ik < cnt, ik + 1.0, 1.0)
    sel = cumm.reshape(st, 1, n) == tgt
    return jnp.where(sel, 1.0, 0.0).reshape(st * k, n)


def _mlp(h, layers):
    for (w, bb, gs, be) in layers:
        z = jax.lax.dot_general(h, w[...], (((1,), (1,)), ((), ())),
                                preferred_element_type=_F32)
        z = (z + bb[...]) * gs[...] + be[...]
        h = jnp.maximum(z, 0.0)
    return h


def _unpack_layers(refs):
    return [tuple(refs[i:i + 4]) for i in range(0, len(refs), 4)]


def _sa1_body(pts_ref, cen_ref, *rest, st, k, r2, dout):
    out_ref = rest[-1]
    layers = _unpack_layers(rest[:-1])
    x = pts_ref[0]
    c = cen_ref[0]
    n = x.shape[1]
    dx = c[:, 0:1] - x[0:1, :]
    dy = c[:, 1:2] - x[1:2, :]
    dz = c[:, 2:3] - x[2:3, :]
    dsq = dx * dx + dy * dy + dz * dz
    mask = dsq <= r2
    cum = _cumsum_lanes(mask.astype(_F32), st, n)
    selm = _select_matrix(cum, mask, st, k, n)
    (w1, b1, g1, e1) = layers[0]
    p1 = jax.lax.dot_general(x, w1[...], (((0,), (1,)), ((), ())),
                             preferred_element_type=_F32)
    g1x = jax.lax.dot_general(selm, p1, (((1,), (0,)), ((), ())),
                              preferred_element_type=_F32)
    c1 = jax.lax.dot_general(c, w1[...], (((1,), (1,)), ((), ())),
                             preferred_element_type=_F32)
    d1 = c1.shape[1]
    c1r = jnp.broadcast_to(c1.reshape(st, 1, d1),
                           (st, k, d1)).reshape(st * k, d1)
    z1 = (g1x - c1r + b1[...]) * g1[...] + e1[...]
    h = _mlp(jnp.maximum(z1, 0.0), layers[1:])
    out_ref[0] = jnp.max(h.reshape(st, k, dout), axis=1)


def _sa2_body(pts_ref, cen_ref, feat_ref, *rest, st, k, r2, dout):
    out_ref = rest[-1]
    layers = _unpack_layers(rest[:-1])
    x = pts_ref[0]
    c = cen_ref[0]
    fb = feat_ref[0]
    n = x.shape[1]
    dx = c[:, 0:1] - x[0:1, :]
    dy = c[:, 1:2] - x[1:2, :]
    dz = c[:, 2:3] - x[2:3, :]
    dsq = dx * dx + dy * dy + dz * dz
    mask = dsq <= r2
    cum = _cumsum_lanes(mask.astype(_F32), st, n)
    selm = _select_matrix(cum, mask, st, k, n)
    gx = jax.lax.dot_general(selm, x, (((1,), (1,)), ((), ())),
                             preferred_element_type=_F32)
    gf = jax.lax.dot_general(selm, fb, (((1,), (0,)), ((), ())),
                             preferred_element_type=_F32)
    crep = jnp.broadcast_to(c.reshape(st, 1, 3), (st, k, 3)).reshape(st * k, 3)
    h = jnp.concatenate([gx - crep, gf], axis=1)
    h = _mlp(h, layers)
    out_ref[0] = jnp.max(h.reshape(st, k, dout), axis=1)


def _sa3_head_body(cen_ref, feat_ref, *rest, nb, ns):
    out_ref = rest[-1]
    wo = rest[-3]
    bo = rest[-2]
    n3 = len(rest) - 3
    layers3 = _unpack_layers(rest[:12])
    layersh = _unpack_layers(rest[12:n3])
    cen = cen_ref[...].reshape(nb * ns, 3)
    fb = feat_ref[...].reshape(nb * ns, -1)
    h = jnp.concatenate([cen, fb], axis=1)
    h = _mlp(h, layers3)
    pooled = jnp.max(h.reshape(nb, ns, -1), axis=1)
    h = _mlp(pooled, layersh)
    logits = jax.lax.dot_general(h, wo[...], (((1,), (1,)), ((), ())),
                                 preferred_element_type=_F32) + bo[...]
    m = jnp.max(logits, axis=1, keepdims=True)
    shifted = logits - m
    lse = jnp.log(jnp.sum(jnp.exp(shifted), axis=1, keepdims=True))
    out_ref[...] = shifted - lse


def _prep_layers(layers):
    s = jnp.sqrt(jnp.asarray(1.0 + _EPS, dtype=_F32))
    out = []
    for (w, b, g, be) in layers:
        out.extend([w, b.reshape(1, -1), (g / s).reshape(1, -1),
                    be.reshape(1, -1)])
    return out


def _const_specs(arrs):
    return [pl.BlockSpec(a.shape, lambda *idx, nd=a.ndim: (0,) * nd)
            for a in arrs]


def _run_fps(x, n_sample):
    b = x.shape[0]
    return pl.pallas_call(
        functools.partial(_fps_body, n_sample=n_sample),
        out_shape=jax.ShapeDtypeStruct((b, n_sample, 3), _F32),
        interpret=_INTERPRET,
    )(x)


def _run_sa(body, pts, cen, feats, wflat, *, st, k, r2, dout):
    b, _, n = pts.shape
    s = cen.shape[1]
    args = [pts, cen] + ([feats] if feats is not None else []) + wflat
    in_specs = [
        pl.BlockSpec((1, 3, n), lambda bi, si: (bi, 0, 0)),
        pl.BlockSpec((1, st, 3), lambda bi, si: (bi, si, 0)),
    ]
    if feats is not None:
        df = feats.shape[2]
        in_specs.append(pl.BlockSpec((1, n, df), lambda bi, si: (bi, 0, 0)))
    in_specs += _const_specs(wflat)
    return pl.pallas_call(
        functools.partial(body, st=st, k=k, r2=r2, dout=dout),
        grid=(b, s // st),
        in_specs=in_specs,
        out_specs=pl.BlockSpec((1, st, dout), lambda bi, si: (bi, si, 0)),
        out_shape=jax.ShapeDtypeStruct((b, s, dout), _F32),
        interpret=_INTERPRET,
    )(*args)


def kernel(coord, params):
    coord = coord.astype(_F32)
    b, _, n = coord.shape

    sa1 = _prep_layers(params["sa1"])
    sa2 = _prep_layers(params["sa2"])
    sa3 = _prep_layers(params["sa3"])
    head = _prep_layers(params["head"])
    wo, bo = params["head_out"]
    bo2 = bo.reshape(1, -1)

    nx1 = _run_fps(coord, 512)
    f1 = _run_sa(_sa1_body, coord, nx1, None, sa1,
                 st=64, k=32, r2=np.float32(0.2 * 0.2), dout=128)

    nx1_t = jnp.transpose(nx1, (0, 2, 1))
    nx2 = _run_fps(nx1_t, 128)
    f2 = _run_sa(_sa2_body, nx1_t, nx2, f1, sa2,
                 st=128, k=64, r2=np.float32(0.4 * 0.4), dout=256)

    logp = pl.pallas_call(
        functools.partial(_sa3_head_body, nb=b, ns=128),
        out_shape=jax.ShapeDtypeStruct((b, 40), _F32),
        interpret=_INTERPRET,
    )(nx2, f2, *sa3, *head, wo, bo2)

    new_xyz3 = jnp.zeros((b, 3, 1), dtype=coord.dtype)
    return logp, new_xyz3


# speedup vs baseline: 17.2004x; 1.0299x over previous
import functools

import jax
import jax.numpy as jnp
import numpy as np
from jax.experimental import pallas as pl

_EPS = 1e-5
_F32 = jnp.float32
_INTERPRET = False


def _fps_body(x_ref, out_ref, *, n_sample):
    b, _, n = x_ref.shape
    xx = x_ref[:, 0, :]
    yy = x_ref[:, 1, :]
    zz = x_ref[:, 2, :]
    xyz3 = jnp.concatenate([xx, yy, zz], axis=0)
    col = jax.lax.broadcasted_iota(jnp.int32, (b, n), 1)
    col3 = jax.lax.broadcasted_iota(jnp.int32, (3 * b, n), 1)

    def step(i, state):
        dist, far = state
        far3 = jnp.concatenate([far, far, far], axis=0)
        picked = jnp.sum(jnp.where(col3 == far3, xyz3, 0.0),
                         axis=1, keepdims=True)
        cx = picked[0:b]
        cy = picked[b:2 * b]
        cz = picked[2 * b:3 * b]
        c3 = jnp.concatenate([cx, cy, cz], axis=1)
        out_ref[:, pl.ds(i, 1), :] = c3.reshape(b, 1, 3)
        dx = xx - cx
        dy = yy - cy
        dz = zz - cz
        d = dx * dx + dy * dy + dz * dz
        dist = jnp.minimum(dist, d)
        m = jnp.max(dist, axis=1, keepdims=True)
        far = jnp.min(col + (n * 2) * (dist < m).astype(jnp.int32),
                      axis=1, keepdims=True)
        return dist, far

    dist0 = jnp.full((b, n), 1e10, dtype=_F32)
    far0 = jnp.zeros((b, 1), dtype=jnp.int32)
    jax.lax.fori_loop(0, n_sample, step, (dist0, far0))


def _cumsum_lanes(mf, st, n, ch=128):
    c = n // ch
    x = mf.reshape(st * c, ch)
    r_i = jax.lax.broadcasted_iota(jnp.int32, (ch, ch), 0)
    c_i = jax.lax.broadcasted_iota(jnp.int32, (ch, ch), 1)
    tri = (r_i <= c_i).astype(_F32)
    loc = jax.lax.dot_general(x, tri, (((1,), (0,)), ((), ())),
                              preferred_element_type=_F32)
    loc = loc.reshape(st, c, ch)
    tot = loc[:, :, ch - 1]
    r2_i = jax.lax.broadcasted_iota(jnp.int32, (c, c), 0)
    c2_i = jax.lax.broadcasted_iota(jnp.int32, (c, c), 1)
    tri_s = (r2_i < c2_i).astype(_F32)
    exc = jax.lax.dot_general(tot, tri_s, (((1,), (0,)), ((), ())),
                              preferred_element_type=_F32)
    cum = loc + exc.reshape(st, c, 1)
    return cum.reshape(st, n)


def _select_matrix(cum, mask, st, k, n):
    cumm = jnp.where(mask, cum, -1.0)
    cnt = jnp.max(cumm, axis=1, keepdims=True).reshape(st, 1, 1)
    ik = jax.lax.broadcasted_iota(jnp.int32, (st, k, 1), 1).astype(_F32)
    tgt = jnp.where(ik < cnt, ik + 1.0, 1.0)
    sel = cumm.reshape(st, 1, n) == tgt
    return jnp.where(sel, 1.0, 0.0).reshape(st * k, n)


def _mlp(h, layers):
    for (w, bb, gs, be) in layers:
        z = jax.lax.dot_general(h, w[...], (((1,), (1,)), ((), ())),
                                preferred_element_type=_F32)
        z = (z + bb[...]) * gs[...] + be[...]
        h = jnp.maximum(z, 0.0)
    return h


def _unpack_layers(refs):
    return [tuple(refs[i:i + 4]) for i in range(0, len(refs), 4)]


def _sa1_body(pts_ref, cen_ref, *rest, st, k, r2, dout):
    out_ref = rest[-1]
    layers = _unpack_layers(rest[:-1])
    x = pts_ref[0]
    c = cen_ref[0]
    n = x.shape[1]
    dx = c[:, 0:1] - x[0:1, :]
    dy = c[:, 1:2] - x[1:2, :]
    dz = c[:, 2:3] - x[2:3, :]
    dsq = dx * dx + dy * dy + dz * dz
    mask = dsq <= r2
    cum = _cumsum_lanes(mask.astype(_F32), st, n)
    selm = _select_matrix(cum, mask, st, k, n)
    (w1, b1, g1, e1) = layers[0]
    p1 = jax.lax.dot_general(x, w1[...], (((0,), (1,)), ((), ())),
                             preferred_element_type=_F32)
    g1x = jax.lax.dot_general(selm, p1, (((1,), (0,)), ((), ())),
                              preferred_element_type=_F32)
    c1 = jax.lax.dot_general(c, w1[...], (((1,), (1,)), ((), ())),
                             preferred_element_type=_F32)
    d1 = c1.shape[1]
    c1r = jnp.broadcast_to(c1.reshape(st, 1, d1),
                           (st, k, d1)).reshape(st * k, d1)
    z1 = (g1x - c1r + b1[...]) * g1[...] + e1[...]
    h = _mlp(jnp.maximum(z1, 0.0), layers[1:])
    out_ref[0] = jnp.max(h.reshape(st, k, dout), axis=1)


def _sa2_body(pts_ref, cen_ref, feat_ref, *rest, st, k, r2, dout):
    out_ref = rest[-1]
    layers = _unpack_layers(rest[:-1])
    x = pts_ref[0]
    c = cen_ref[0]
    fb = feat_ref[0]
    n = x.shape[1]
    dx = c[:, 0:1] - x[0:1, :]
    dy = c[:, 1:2] - x[1:2, :]
    dz = c[:, 2:3] - x[2:3, :]
    dsq = dx * dx + dy * dy + dz * dz
    mask = dsq <= r2
    cum = _cumsum_lanes(mask.astype(_F32), st, n)
    selm = _select_matrix(cum, mask, st, k, n)
    (w1, b1, g1, e1) = layers[0]
    wx = w1[:, 0:3]
    wf = w1[:, 3:]
    p1 = (jax.lax.dot_general(x, wx, (((0,), (1,)), ((), ())),
                              preferred_element_type=_F32)
          + jax.lax.dot_general(fb, wf, (((1,), (1,)), ((), ())),
                                preferred_element_type=_F32))
    g1x = jax.lax.dot_general(selm, p1, (((1,), (0,)), ((), ())),
                              preferred_element_type=_F32)
    c1 = jax.lax.dot_general(c, wx, (((1,), (1,)), ((), ())),
                             preferred_element_type=_F32)
    d1 = c1.shape[1]
    c1r = jnp.broadcast_to(c1.reshape(st, 1, d1),
                           (st, k, d1)).reshape(st * k, d1)
    z1 = (g1x - c1r + b1[...]) * g1[...] + e1[...]
    h = _mlp(jnp.maximum(z1, 0.0), layers[1:])
    out_ref[0] = jnp.max(h.reshape(st, k, dout), axis=1)


def _sa3_head_body(cen_ref, feat_ref, *rest, nb, ns):
    out_ref = rest[-1]
    wo = rest[-3]
    bo = rest[-2]
    n3 = len(rest) - 3
    layers3 = _unpack_layers(rest[:12])
    layersh = _unpack_layers(rest[12:n3])
    cen = cen_ref[...].reshape(nb * ns, 3)
    fb = feat_ref[...].reshape(nb * ns, -1)
    h = jnp.concatenate([cen, fb], axis=1)
    h = _mlp(h, layers3)
    pooled = jnp.max(h.reshape(nb, ns, -1), axis=1)
    h = _mlp(pooled, layersh)
    logits = jax.lax.dot_general(h, wo[...], (((1,), (1,)), ((), ())),
                                 preferred_element_type=_F32) + bo[...]
    m = jnp.max(logits, axis=1, keepdims=True)
    shifted = logits - m
    lse = jnp.log(jnp.sum(jnp.exp(shifted), axis=1, keepdims=True))
    out_ref[...] = shifted - lse


def _prep_layers(layers):
    s = jnp.sqrt(jnp.asarray(1.0 + _EPS, dtype=_F32))
    out = []
    for (w, b, g, be) in layers:
        out.extend([w, b.reshape(1, -1), (g / s).reshape(1, -1),
                    be.reshape(1, -1)])
    return out


def _const_specs(arrs):
    return [pl.BlockSpec(a.shape, lambda *idx, nd=a.ndim: (0,) * nd)
            for a in arrs]


def _run_fps(x, n_sample):
    b = x.shape[0]
    return pl.pallas_call(
        functools.partial(_fps_body, n_sample=n_sample),
        out_shape=jax.ShapeDtypeStruct((b, n_sample, 3), _F32),
        interpret=_INTERPRET,
    )(x)


def _run_sa(body, pts, cen, feats, wflat, *, st, k, r2, dout):
    b, _, n = pts.shape
    s = cen.shape[1]
    args = [pts, cen] + ([feats] if feats is not None else []) + wflat
    in_specs = [
        pl.BlockSpec((1, 3, n), lambda bi, si: (bi, 0, 0)),
        pl.BlockSpec((1, st, 3), lambda bi, si: (bi, si, 0)),
    ]
    if feats is not None:
        df = feats.shape[2]
        in_specs.append(pl.BlockSpec((1, n, df), lambda bi, si: (bi, 0, 0)))
    in_specs += _const_specs(wflat)
    return pl.pallas_call(
        functools.partial(body, st=st, k=k, r2=r2, dout=dout),
        grid=(b, s // st),
        in_specs=in_specs,
        out_specs=pl.BlockSpec((1, st, dout), lambda bi, si: (bi, si, 0)),
        out_shape=jax.ShapeDtypeStruct((b, s, dout), _F32),
        interpret=_INTERPRET,
    )(*args)


def kernel(coord, params):
    coord = coord.astype(_F32)
    b, _, n = coord.shape

    sa1 = _prep_layers(params["sa1"])
    sa2 = _prep_layers(params["sa2"])
    sa3 = _prep_layers(params["sa3"])
    head = _prep_layers(params["head"])
    wo, bo = params["head_out"]
    bo2 = bo.reshape(1, -1)

    nx1 = _run_fps(coord, 512)
    f1 = _run_sa(_sa1_body, coord, nx1, None, sa1,
                 st=64, k=32, r2=np.float32(0.2 * 0.2), dout=128)

    nx1_t = jnp.transpose(nx1, (0, 2, 1))
    nx2 = _run_fps(nx1_t, 128)
    f2 = _run_sa(_sa2_body, nx1_t, nx2, f1, sa2,
                 st=128, k=64, r2=np.float32(0.4 * 0.4), dout=256)

    logp = pl.pallas_call(
        functools.partial(_sa3_head_body, nb=b, ns=128),
        out_shape=jax.ShapeDtypeStruct((b, 40), _F32),
        interpret=_INTERPRET,
    )(nx2, f2, *sa3, *head, wo, bo2)

    new_xyz3 = jnp.zeros((b, 3, 1), dtype=coord.dtype)
    return logp, new_xyz3
